# baseline (speedup 1.0000x reference)
import functools

import jax
import jax.numpy as jnp
from jax.experimental import pallas as pl
from jax.experimental.pallas import tpu as pltpu
from jax.experimental.pallas import tpu_sc as plsc

_EPS = 1e-05
_LN_EPS = 1e-05

_T = 4096
_H = 1024
_I = 4096
_E = 8
_EL = 128
_TILE = 256
_NT = 40
_NP = _NT * _TILE
_IC = 512
_NJ = _I // _IC

_NW = 32
_CH = 32


def _router_body(x_ref, g_ref, b_ref, gw_ref, w_ref, i_ref):
    x = x_ref[...]
    mu = jnp.mean(x, axis=1, keepdims=True)
    xc = x - mu
    var = jnp.mean(xc * xc, axis=1, keepdims=True)
    xn = xc * jax.lax.rsqrt(var + _LN_EPS) * g_ref[...] + b_ref[...]
    logits = jax.lax.dot_general(xn, gw_ref[...], (((1,), (1,)), ((), ())),
                                 preferred_element_type=jnp.float32)
    lane = jax.lax.broadcasted_iota(jnp.int32, logits.shape, 1)
    valid = lane < _E
    logits = jnp.where(valid, logits, -1e30)
    m = jnp.max(logits, axis=1, keepdims=True)
    p = jnp.exp(logits - m)
    p = jnp.where(valid, p, 0.0)
    probs = p / jnp.sum(p, axis=1, keepdims=True)
    p0 = jnp.max(probs, axis=1, keepdims=True)
    i0 = jnp.min(jnp.where(probs >= p0, lane, _EL), axis=1, keepdims=True)
    probs2 = jnp.where(lane == i0, -1.0, probs)
    p1 = jnp.max(probs2, axis=1, keepdims=True)
    i1 = jnp.min(jnp.where(probs2 >= p1, lane, _EL), axis=1, keepdims=True)
    s = p0 + p1 + _EPS
    w0 = p0 / s
    w1 = p1 / s
    w_ref[...] = jnp.where(lane == 0, w0, jnp.where(lane == 1, w1, 0.0))
    i_ref[...] = jnp.where(lane == 0, i0, jnp.where(lane == 1, i1, 0))


def _router(x_flat, gamma, beta, gw_pad):
    tt = 256
    return pl.pallas_call(
        _router_body,
        grid=(_T // tt,),
        in_specs=[
            pl.BlockSpec((tt, _H), lambda i: (i, 0)),
            pl.BlockSpec((1, _H), lambda i: (0, 0)),
            pl.BlockSpec((1, _H), lambda i: (0, 0)),
            pl.BlockSpec((_EL, _H), lambda i: (0, 0)),
        ],
        out_specs=[
            pl.BlockSpec((tt, _EL), lambda i: (i, 0)),
            pl.BlockSpec((tt, _EL), lambda i: (i, 0)),
        ],
        out_shape=[
            jax.ShapeDtypeStruct((_T, _EL), jnp.float32),
            jax.ShapeDtypeStruct((_T, _EL), jnp.int32),
        ],
    )(x_flat, gamma, beta, gw_pad)


def _ridx_body(i0_ref, i1_ref, pos_ref, te_ref):
    erow = jax.lax.broadcasted_iota(jnp.int32, (_E, _T), 0)
    oh0 = (i0_ref[...] == erow).astype(jnp.int32)
    oh1 = (i1_ref[...] == erow).astype(jnp.int32)
    li = jax.lax.broadcasted_iota(jnp.int32, (_E, _T), 1)

    def lane_cumsum(a):
        s = 1
        while s < _T:
            sh = pltpu.roll(a, s, axis=1)
            a = a + jnp.where(li >= s, sh, 0)
            s *= 2
        return a

    c0 = lane_cumsum(oh0)
    c1 = lane_cumsum(oh1)
    tot0 = jnp.sum(oh0, axis=1, keepdims=True)
    counts = tot0 + jnp.sum(oh1, axis=1, keepdims=True)
    nt = (counts + (_TILE - 1)) // _TILE
    inc = nt
    s = 1
    while s < _E:
        inc = inc + jnp.concatenate(
            [jnp.zeros((s, 1), jnp.int32), inc[:-s]], axis=0)
        s *= 2
    ts = inc - nt
    start = ts * _TILE
    pos0 = jnp.sum(oh0 * (start + c0 - 1), axis=0, keepdims=True)
    pos1 = jnp.sum(oh1 * (start + tot0 + c1 - 1), axis=0, keepdims=True)
    ri = jax.lax.broadcasted_iota(jnp.int32, (_E, _T), 0)
    pos_ref[...] = jnp.where(ri == 0, pos0, jnp.where(ri == 1, pos1, 0))
    ti = jax.lax.broadcasted_iota(jnp.int32, (_E, _EL), 1)
    te = jnp.sum((ts <= ti).astype(jnp.int32), axis=0, keepdims=True) - 1
    te = jnp.clip(te, 0, _E - 1)
    te_ref[...] = jnp.broadcast_to(te, (_E, _EL))


def _ridx(i0r, i1r):
    return pl.pallas_call(
        _ridx_body,
        out_shape=[
            jax.ShapeDtypeStruct((_E, _T), jnp.int32),
            jax.ShapeDtypeStruct((_E, _EL), jnp.int32),
        ],
    )(i0r, i1r)


def _sc_dispatch(x_flat, pos_flat):
    mesh = plsc.VectorSubcoreMesh(core_axis_name="c", subcore_axis_name="s")
    tpw = _T // _NW

    @functools.partial(
        pl.kernel,
        out_type=jax.ShapeDtypeStruct((_NP, _H), jnp.float32),
        mesh=mesh,
        scratch_types=[
            pltpu.VMEM((_CH,), jnp.int32),
            pltpu.VMEM((_CH, _H), jnp.float32),
            pltpu.SemaphoreType.DMA,
        ],
    )
    def _k(x_hbm, pos_hbm, xd_hbm, idx_v, rows_v, sem):
        wid = jax.lax.axis_index("s") * 2 + jax.lax.axis_index("c")
        base = wid * tpw
        for k in range(2):
            koff = k * _T

            @pl.loop(0, tpw, step=_CH)
            def _(c):
                b = base + c
                pltpu.sync_copy(pos_hbm.at[pl.ds(koff + b, _CH)], idx_v)
                pltpu.sync_copy(x_hbm.at[pl.ds(b, _CH)], rows_v)
                pltpu.async_copy(rows_v, xd_hbm.at[idx_v], sem).wait()

    return _k(x_flat, pos_flat)


def _gffn_body(te_ref, xd_ref, eg_ref, eu_ref, ed_ref, y_ref):
    j = pl.program_id(0)
    i = pl.program_id(1)
    x = xd_ref[...].astype(jnp.bfloat16)
    g = eg_ref[0]
    u = eu_ref[0]
    d = ed_ref[0]
    a = jax.lax.dot_general(x, g, (((1,), (1,)), ((), ())),
                            preferred_element_type=jnp.float32)
    bb = jax.lax.dot_general(x, u, (((1,), (1,)), ((), ())),
                             preferred_element_type=jnp.float32)
    h = (a * jax.lax.logistic(a) * bb).astype(jnp.bfloat16)
    yj = jax.lax.dot_general(h, d, (((1,), (1,)), ((), ())),
                             preferred_element_type=jnp.float32)
    sl = pl.ds(i * _TILE, _TILE)

    @pl.when(j == 0)
    def _():
        y_ref[sl, :] = yj.astype(y_ref.dtype)

    @pl.when(j != 0)
    def _():
        y_ref[sl, :] += yj.astype(y_ref.dtype)


def _gffn(te, xd, eg, eu, ed):
    grid_spec = pltpu.PrefetchScalarGridSpec(
        num_scalar_prefetch=1,
        grid=(_NJ, _NT),
        in_specs=[
            pl.BlockSpec((_TILE, _H), lambda j, i, te: (i, 0)),
            pl.BlockSpec((1, _IC, _H), lambda j, i, te: (te[i], j, 0)),
            pl.BlockSpec((1, _IC, _H), lambda j, i, te: (te[i], j, 0)),
            pl.BlockSpec((1, _H, _IC), lambda j, i, te: (te[i], 0, j)),
        ],
        out_specs=pl.BlockSpec((_NP, _H), lambda j, i, te: (0, 0)),
    )
    return pl.pallas_call(
        _gffn_body,
        grid_spec=grid_spec,
        out_shape=jax.ShapeDtypeStruct((_NP, _H), jnp.float32),
        compiler_params=pltpu.CompilerParams(
            dimension_semantics=("arbitrary", "arbitrary")),
    )(te, xd, eg, eu, ed)


def _sffn_body(x_ref, sg_ref, su_ref, sd_ref, y_ref):
    j = pl.program_id(0)
    i = pl.program_id(1)
    x = x_ref[...]
    a = jax.lax.dot_general(x, sg_ref[...], (((1,), (1,)), ((), ())),
                            preferred_element_type=jnp.float32)
    bb = jax.lax.dot_general(x, su_ref[...], (((1,), (1,)), ((), ())),
                             preferred_element_type=jnp.float32)
    h = (a * jax.lax.logistic(a) * bb).astype(jnp.bfloat16)
    yj = jax.lax.dot_general(h, sd_ref[...], (((1,), (1,)), ((), ())),
                             preferred_element_type=jnp.float32)
    sl = pl.ds(i * _TILE, _TILE)

    @pl.when(j == 0)
    def _():
        y_ref[sl, :] = yj

    @pl.when(j != 0)
    def _():
        y_ref[sl, :] += yj


def _sffn(x_flat, sg, su, sd):
    return pl.pallas_call(
        _sffn_body,
        grid=(_NJ, _T // _TILE),
        in_specs=[
            pl.BlockSpec((_TILE, _H), lambda j, i: (i, 0)),
            pl.BlockSpec((_IC, _H), lambda j, i: (j, 0)),
            pl.BlockSpec((_IC, _H), lambda j, i: (j, 0)),
            pl.BlockSpec((_H, _IC), lambda j, i: (0, j)),
        ],
        out_specs=pl.BlockSpec((_T, _H), lambda j, i: (0, 0)),
        out_shape=jax.ShapeDtypeStruct((_T, _H), jnp.float32),
        compiler_params=pltpu.CompilerParams(
            dimension_semantics=("arbitrary", "arbitrary")),
    )(x_flat, sg, su, sd)


def _sc_combine(y, pos_flat):
    mesh = plsc.VectorSubcoreMesh(core_axis_name="c", subcore_axis_name="s")
    tpw = _T // _NW

    @functools.partial(
        pl.kernel,
        out_type=[
            jax.ShapeDtypeStruct((_T, _H), jnp.float32),
            jax.ShapeDtypeStruct((_T, _H), jnp.float32),
        ],
        mesh=mesh,
        scratch_types=[
            pltpu.VMEM((_CH,), jnp.int32),
            pltpu.VMEM((_CH, _H), jnp.float32),
            pltpu.SemaphoreType.DMA,
        ],
    )
    def _k(y_hbm, pos_hbm, g0_hbm, g1_hbm, idx_v, rows_v, sem):
        wid = jax.lax.axis_index("s") * 2 + jax.lax.axis_index("c")
        base = wid * tpw
        for k in range(2):
            koff = k * _T
            out = g0_hbm if k == 0 else g1_hbm

            @pl.loop(0, tpw, step=_CH)
            def _(c):
                b = base + c
                pltpu.sync_copy(pos_hbm.at[pl.ds(koff + b, _CH)], idx_v)
                pltpu.async_copy(y_hbm.at[idx_v], rows_v, sem).wait()
                pltpu.sync_copy(rows_v, out.at[pl.ds(b, _CH)])

    return _k(y, pos_flat)


def _fin_body(sh_ref, g0_ref, g1_ref, w_ref, o_ref):
    w0 = w_ref[:, 0:1]
    w1 = w_ref[:, 1:2]
    o_ref[...] = (sh_ref[...]
                  + w0 * g0_ref[...].astype(jnp.float32)
                  + w1 * g1_ref[...].astype(jnp.float32))


def _final(sh, g0, g1, wout):
    return pl.pallas_call(
        _fin_body,
        grid=(_T // _TILE,),
        in_specs=[
            pl.BlockSpec((_TILE, _H), lambda i: (i, 0)),
            pl.BlockSpec((_TILE, _H), lambda i: (i, 0)),
            pl.BlockSpec((_TILE, _H), lambda i: (i, 0)),
            pl.BlockSpec((_TILE, _EL), lambda i: (i, 0)),
        ],
        out_specs=pl.BlockSpec((_TILE, _H), lambda i: (i, 0)),
        out_shape=jax.ShapeDtypeStruct((_T, _H), jnp.float32),
    )(sh, g0, g1, wout)


def kernel(x, ln_gamma, ln_beta, gate_w, eg, eu, ed, sg, su, sd):
    b, s, h = x.shape
    x_flat = x.reshape(-1, h)
    gw_pad = jnp.zeros((_EL, _H), jnp.float32).at[:_E].set(gate_w)
    wout, iout = _router(x_flat, ln_gamma.reshape(1, _H),
                         ln_beta.reshape(1, _H), gw_pad)
    i0r = iout[:, 0].reshape(1, _T)
    i1r = iout[:, 1].reshape(1, _T)
    pos_out, te_out = _ridx(i0r, i1r)
    pos_flat = pos_out[:2].reshape(-1)
    te = te_out[0]
    xd = _sc_dispatch(x_flat, pos_flat)
    y = _gffn(te, xd, eg.astype(jnp.bfloat16), eu.astype(jnp.bfloat16),
              ed.astype(jnp.bfloat16))
    sh = _sffn(x_flat.astype(jnp.bfloat16), sg.astype(jnp.bfloat16),
               su.astype(jnp.bfloat16), sd.astype(jnp.bfloat16))
    g0, g1 = _sc_combine(y, pos_flat)
    out = _final(sh, g0, g1, wout)
    return out.reshape(b, s, h)


# speedup vs baseline: 1.1802x; 1.1802x over previous
import functools

import jax
import jax.numpy as jnp
from jax.experimental import pallas as pl
from jax.experimental.pallas import tpu as pltpu
from jax.experimental.pallas import tpu_sc as plsc

_EPS = 1e-05
_LN_EPS = 1e-05

_T = 4096
_H = 1024
_I = 4096
_E = 8
_EL = 128
_TILE = 256
_NT = 40
_NP = _NT * _TILE
_IC = 512
_NJ = _I // _IC

_NW = 32
_CH = 32


def _router_body(x_ref, g_ref, b_ref, gw_ref, w_ref, i_ref):
    x = x_ref[...]
    mu = jnp.mean(x, axis=1, keepdims=True)
    xc = x - mu
    var = jnp.mean(xc * xc, axis=1, keepdims=True)
    xn = xc * jax.lax.rsqrt(var + _LN_EPS) * g_ref[...] + b_ref[...]
    logits = jax.lax.dot_general(xn, gw_ref[...], (((1,), (1,)), ((), ())),
                                 preferred_element_type=jnp.float32)
    lane = jax.lax.broadcasted_iota(jnp.int32, logits.shape, 1)
    valid = lane < _E
    logits = jnp.where(valid, logits, -1e30)
    m = jnp.max(logits, axis=1, keepdims=True)
    p = jnp.exp(logits - m)
    p = jnp.where(valid, p, 0.0)
    probs = p / jnp.sum(p, axis=1, keepdims=True)
    p0 = jnp.max(probs, axis=1, keepdims=True)
    i0 = jnp.min(jnp.where(probs >= p0, lane, _EL), axis=1, keepdims=True)
    probs2 = jnp.where(lane == i0, -1.0, probs)
    p1 = jnp.max(probs2, axis=1, keepdims=True)
    i1 = jnp.min(jnp.where(probs2 >= p1, lane, _EL), axis=1, keepdims=True)
    s = p0 + p1 + _EPS
    w0 = p0 / s
    w1 = p1 / s
    w_ref[...] = jnp.where(lane == 0, w0, jnp.where(lane == 1, w1, 0.0))
    i_ref[...] = jnp.where(lane == 0, i0, jnp.where(lane == 1, i1, 0))


def _router(x_flat, gamma, beta, gw_pad):
    tt = 256
    return pl.pallas_call(
        _router_body,
        grid=(_T // tt,),
        in_specs=[
            pl.BlockSpec((tt, _H), lambda i: (i, 0)),
            pl.BlockSpec((1, _H), lambda i: (0, 0)),
            pl.BlockSpec((1, _H), lambda i: (0, 0)),
            pl.BlockSpec((_EL, _H), lambda i: (0, 0)),
        ],
        out_specs=[
            pl.BlockSpec((tt, _EL), lambda i: (i, 0)),
            pl.BlockSpec((tt, _EL), lambda i: (i, 0)),
        ],
        out_shape=[
            jax.ShapeDtypeStruct((_T, _EL), jnp.float32),
            jax.ShapeDtypeStruct((_T, _EL), jnp.int32),
        ],
    )(x_flat, gamma, beta, gw_pad)


def _ridx_body(i0_ref, i1_ref, pos_ref, te_ref):
    erow = jax.lax.broadcasted_iota(jnp.int32, (_E, _T), 0)
    oh0 = (i0_ref[...] == erow).astype(jnp.int32)
    oh1 = (i1_ref[...] == erow).astype(jnp.int32)
    li = jax.lax.broadcasted_iota(jnp.int32, (_E, _T), 1)

    def lane_cumsum(a):
        s = 1
        while s < _T:
            sh = pltpu.roll(a, s, axis=1)
            a = a + jnp.where(li >= s, sh, 0)
            s *= 2
        return a

    c0 = lane_cumsum(oh0)
    c1 = lane_cumsum(oh1)
    tot0 = jnp.sum(oh0, axis=1, keepdims=True)
    counts = tot0 + jnp.sum(oh1, axis=1, keepdims=True)
    nt = (counts + (_TILE - 1)) // _TILE
    inc = nt
    s = 1
    while s < _E:
        inc = inc + jnp.concatenate(
            [jnp.zeros((s, 1), jnp.int32), inc[:-s]], axis=0)
        s *= 2
    ts = inc - nt
    start = ts * _TILE
    pos0 = jnp.sum(oh0 * (start + c0 - 1), axis=0, keepdims=True)
    pos1 = jnp.sum(oh1 * (start + tot0 + c1 - 1), axis=0, keepdims=True)
    ri = jax.lax.broadcasted_iota(jnp.int32, (_E, _T), 0)
    pos_ref[...] = jnp.where(ri == 0, pos0, jnp.where(ri == 1, pos1, 0))
    ti = jax.lax.broadcasted_iota(jnp.int32, (_E, _EL), 1)
    te = jnp.sum((ts <= ti).astype(jnp.int32), axis=0, keepdims=True) - 1
    te = jnp.clip(te, 0, _E - 1)
    ntot = jnp.sum(nt, axis=0, keepdims=True)
    ri2 = jax.lax.broadcasted_iota(jnp.int32, (_E, _EL), 0)
    te_ref[...] = jnp.where(ri2 == 1, ntot, te)


def _ridx(i0r, i1r):
    return pl.pallas_call(
        _ridx_body,
        out_shape=[
            jax.ShapeDtypeStruct((_E, _T), jnp.int32),
            jax.ShapeDtypeStruct((_E, _EL), jnp.int32),
        ],
    )(i0r, i1r)


def _sc_dispatch(x_flat, pos_flat):
    mesh = plsc.VectorSubcoreMesh(core_axis_name="c", subcore_axis_name="s")
    tpw = _T // _NW

    @functools.partial(
        pl.kernel,
        out_type=jax.ShapeDtypeStruct((_NP, _H), jnp.float32),
        mesh=mesh,
        scratch_types=[
            pltpu.VMEM((_CH,), jnp.int32),
            pltpu.VMEM((_CH, _H), jnp.float32),
            pltpu.SemaphoreType.DMA,
        ],
    )
    def _k(x_hbm, pos_hbm, xd_hbm, idx_v, rows_v, sem):
        wid = jax.lax.axis_index("s") * 2 + jax.lax.axis_index("c")
        base = wid * tpw
        for k in range(2):
            koff = k * _T

            @pl.loop(0, tpw, step=_CH)
            def _(c):
                b = base + c
                pltpu.sync_copy(pos_hbm.at[pl.ds(koff + b, _CH)], idx_v)
                pltpu.sync_copy(x_hbm.at[pl.ds(b, _CH)], rows_v)
                pltpu.async_copy(rows_v, xd_hbm.at[idx_v], sem).wait()

    return _k(x_flat, pos_flat)


def _gffn_body(te_ref, xd_ref, eg_ref, eu_ref, ed_ref, y_ref):
    j = pl.program_id(0)
    i = pl.program_id(1)
    n_active = te_ref[_EL]

    @pl.when(i < n_active)
    def _():
        x = xd_ref[...].astype(jnp.float32)
        g = eg_ref[0]
        u = eu_ref[0]
        d = ed_ref[0]
        a = jax.lax.dot_general(x, g, (((1,), (1,)), ((), ())),
                                preferred_element_type=jnp.float32)
        bb = jax.lax.dot_general(x, u, (((1,), (1,)), ((), ())),
                                 preferred_element_type=jnp.float32)
        h = a * jax.lax.logistic(a) * bb
        yj = jax.lax.dot_general(h, d, (((1,), (1,)), ((), ())),
                                 preferred_element_type=jnp.float32)
        sl = pl.ds(i * _TILE, _TILE)

        @pl.when(j == 0)
        def _():
            y_ref[sl, :] = yj.astype(y_ref.dtype)

        @pl.when(j != 0)
        def _():
            y_ref[sl, :] += yj.astype(y_ref.dtype)


def _gffn(te, xd, eg, eu, ed):
    grid_spec = pltpu.PrefetchScalarGridSpec(
        num_scalar_prefetch=1,
        grid=(_NJ, _NT),
        in_specs=[
            pl.BlockSpec((_TILE, _H), lambda j, i, te: (i, 0)),
            pl.BlockSpec((1, _IC, _H), lambda j, i, te: (te[i], j, 0)),
            pl.BlockSpec((1, _IC, _H), lambda j, i, te: (te[i], j, 0)),
            pl.BlockSpec((1, _H, _IC), lambda j, i, te: (te[i], 0, j)),
        ],
        out_specs=pl.BlockSpec((_NP, _H), lambda j, i, te: (0, 0)),
    )
    return pl.pallas_call(
        _gffn_body,
        grid_spec=grid_spec,
        out_shape=jax.ShapeDtypeStruct((_NP, _H), jnp.float32),
        compiler_params=pltpu.CompilerParams(
            dimension_semantics=("arbitrary", "arbitrary")),
    )(te, xd, eg, eu, ed)


def _sffn_body(x_ref, sg_ref, su_ref, sd_ref, y_ref):
    j = pl.program_id(0)
    i = pl.program_id(1)
    x = x_ref[pl.ds(i * _TILE, _TILE), :].astype(jnp.float32)
    a = jax.lax.dot_general(x, sg_ref[...], (((1,), (1,)), ((), ())),
                            preferred_element_type=jnp.float32)
    bb = jax.lax.dot_general(x, su_ref[...], (((1,), (1,)), ((), ())),
                             preferred_element_type=jnp.float32)
    h = a * jax.lax.logistic(a) * bb
    yj = jax.lax.dot_general(h, sd_ref[...], (((1,), (1,)), ((), ())),
                             preferred_element_type=jnp.float32)
    sl = pl.ds(i * _TILE, _TILE)

    @pl.when(j == 0)
    def _():
        y_ref[sl, :] = yj

    @pl.when(j != 0)
    def _():
        y_ref[sl, :] += yj


def _sffn(x_flat, sg, su, sd):
    return pl.pallas_call(
        _sffn_body,
        grid=(_NJ, _T // _TILE),
        in_specs=[
            pl.BlockSpec((_T, _H), lambda j, i: (0, 0)),
            pl.BlockSpec((_IC, _H), lambda j, i: (j, 0)),
            pl.BlockSpec((_IC, _H), lambda j, i: (j, 0)),
            pl.BlockSpec((_H, _IC), lambda j, i: (0, j)),
        ],
        out_specs=pl.BlockSpec((_T, _H), lambda j, i: (0, 0)),
        out_shape=jax.ShapeDtypeStruct((_T, _H), jnp.float32),
        compiler_params=pltpu.CompilerParams(
            dimension_semantics=("arbitrary", "arbitrary")),
    )(x_flat, sg, su, sd)


def _sc_combine(y, pos_flat):
    mesh = plsc.VectorSubcoreMesh(core_axis_name="c", subcore_axis_name="s")
    tpw = _T // _NW

    @functools.partial(
        pl.kernel,
        out_type=[
            jax.ShapeDtypeStruct((_T, _H), jnp.float32),
            jax.ShapeDtypeStruct((_T, _H), jnp.float32),
        ],
        mesh=mesh,
        scratch_types=[
            pltpu.VMEM((_CH,), jnp.int32),
            pltpu.VMEM((_CH, _H), jnp.float32),
            pltpu.SemaphoreType.DMA,
        ],
    )
    def _k(y_hbm, pos_hbm, g0_hbm, g1_hbm, idx_v, rows_v, sem):
        wid = jax.lax.axis_index("s") * 2 + jax.lax.axis_index("c")
        base = wid * tpw
        for k in range(2):
            koff = k * _T
            out = g0_hbm if k == 0 else g1_hbm

            @pl.loop(0, tpw, step=_CH)
            def _(c):
                b = base + c
                pltpu.sync_copy(pos_hbm.at[pl.ds(koff + b, _CH)], idx_v)
                pltpu.async_copy(y_hbm.at[idx_v], rows_v, sem).wait()
                pltpu.sync_copy(rows_v, out.at[pl.ds(b, _CH)])

    return _k(y, pos_flat)


def _fin_body(sh_ref, g0_ref, g1_ref, w_ref, o_ref):
    w0 = w_ref[:, 0:1]
    w1 = w_ref[:, 1:2]
    o_ref[...] = (sh_ref[...]
                  + w0 * g0_ref[...].astype(jnp.float32)
                  + w1 * g1_ref[...].astype(jnp.float32))


def _final(sh, g0, g1, wout):
    return pl.pallas_call(
        _fin_body,
        grid=(_T // _TILE,),
        in_specs=[
            pl.BlockSpec((_TILE, _H), lambda i: (i, 0)),
            pl.BlockSpec((_TILE, _H), lambda i: (i, 0)),
            pl.BlockSpec((_TILE, _H), lambda i: (i, 0)),
            pl.BlockSpec((_TILE, _EL), lambda i: (i, 0)),
        ],
        out_specs=pl.BlockSpec((_TILE, _H), lambda i: (i, 0)),
        out_shape=jax.ShapeDtypeStruct((_T, _H), jnp.float32),
    )(sh, g0, g1, wout)


def kernel(x, ln_gamma, ln_beta, gate_w, eg, eu, ed, sg, su, sd):
    b, s, h = x.shape
    x_flat = x.reshape(-1, h)
    gw_pad = jnp.zeros((_EL, _H), jnp.float32).at[:_E].set(gate_w)
    wout, iout = _router(x_flat, ln_gamma.reshape(1, _H),
                         ln_beta.reshape(1, _H), gw_pad)
    i0r = iout[:, 0].reshape(1, _T)
    i1r = iout[:, 1].reshape(1, _T)
    pos_out, te_out = _ridx(i0r, i1r)
    pos_flat = pos_out[:2].reshape(-1)
    te = te_out[:2].reshape(-1)
    xd = _sc_dispatch(x_flat, pos_flat)
    y = _gffn(te, xd.astype(jnp.bfloat16), eg, eu, ed)
    sh = _sffn(x_flat.astype(jnp.bfloat16), sg, su, sd)
    g0, g1 = _sc_combine(y, pos_flat)
    out = _final(sh, g0, g1, wout)
    return out.reshape(b, s, h)


# speedup vs baseline: 1.5492x; 1.3127x over previous
import functools

import jax
import jax.numpy as jnp
from jax.experimental import pallas as pl
from jax.experimental.pallas import tpu as pltpu
from jax.experimental.pallas import tpu_sc as plsc

_EPS = 1e-05
_LN_EPS = 1e-05

_T = 4096
_H = 1024
_I = 4096
_E = 8
_EL = 128
_TILE = 512
_NT = 24
_NQ = 4
_NH = _NT // _NQ
_NP = _NT * _TILE
_IC = 1024
_NJ = _I // _IC

_NW = 32
_CH = 32


def _router_body(x_ref, g_ref, b_ref, gw_ref, w_ref, i_ref):
    x = x_ref[...]
    mu = jnp.mean(x, axis=1, keepdims=True)
    xc = x - mu
    var = jnp.mean(xc * xc, axis=1, keepdims=True)
    xn = xc * jax.lax.rsqrt(var + _LN_EPS) * g_ref[...] + b_ref[...]
    logits = jax.lax.dot_general(xn, gw_ref[...], (((1,), (1,)), ((), ())),
                                 preferred_element_type=jnp.float32)
    lane = jax.lax.broadcasted_iota(jnp.int32, logits.shape, 1)
    valid = lane < _E
    logits = jnp.where(valid, logits, -1e30)
    m = jnp.max(logits, axis=1, keepdims=True)
    p = jnp.exp(logits - m)
    p = jnp.where(valid, p, 0.0)
    probs = p / jnp.sum(p, axis=1, keepdims=True)
    p0 = jnp.max(probs, axis=1, keepdims=True)
    i0 = jnp.min(jnp.where(probs >= p0, lane, _EL), axis=1, keepdims=True)
    probs2 = jnp.where(lane == i0, -1.0, probs)
    p1 = jnp.max(probs2, axis=1, keepdims=True)
    i1 = jnp.min(jnp.where(probs2 >= p1, lane, _EL), axis=1, keepdims=True)
    s = p0 + p1 + _EPS
    w0 = p0 / s
    w1 = p1 / s
    w_ref[...] = jnp.where(lane == 0, w0, jnp.where(lane == 1, w1, 0.0))
    i_ref[...] = jnp.where(lane == 0, i0, jnp.where(lane == 1, i1, 0))


def _router(x_flat, gamma, beta, gw_pad):
    tt = 512
    return pl.pallas_call(
        _router_body,
        grid=(_T // tt,),
        in_specs=[
            pl.BlockSpec((tt, _H), lambda i: (i, 0)),
            pl.BlockSpec((1, _H), lambda i: (0, 0)),
            pl.BlockSpec((1, _H), lambda i: (0, 0)),
            pl.BlockSpec((_EL, _H), lambda i: (0, 0)),
        ],
        out_specs=[
            pl.BlockSpec((tt, _EL), lambda i: (i, 0)),
            pl.BlockSpec((tt, _EL), lambda i: (i, 0)),
        ],
        out_shape=[
            jax.ShapeDtypeStruct((_T, _EL), jnp.float32),
            jax.ShapeDtypeStruct((_T, _EL), jnp.int32),
        ],
    )(x_flat, gamma, beta, gw_pad)


def _ridx_body(i0_ref, i1_ref, pos_ref, te_ref):
    erow = jax.lax.broadcasted_iota(jnp.int32, (_E, _T), 0)
    oh0 = (i0_ref[...] == erow).astype(jnp.int32)
    oh1 = (i1_ref[...] == erow).astype(jnp.int32)
    li = jax.lax.broadcasted_iota(jnp.int32, (_E, _T), 1)

    def lane_cumsum(a):
        s = 1
        while s < _T:
            sh = pltpu.roll(a, s, axis=1)
            a = a + jnp.where(li >= s, sh, 0)
            s *= 2
        return a

    c0 = lane_cumsum(oh0)
    c1 = lane_cumsum(oh1)
    tot0 = jnp.sum(oh0, axis=1, keepdims=True)
    counts = tot0 + jnp.sum(oh1, axis=1, keepdims=True)
    nt = (counts + (_TILE - 1)) // _TILE
    inc = nt
    s = 1
    while s < _E:
        inc = inc + jnp.concatenate(
            [jnp.zeros((s, 1), jnp.int32), inc[:-s]], axis=0)
        s *= 2
    ts = inc - nt
    start = ts * _TILE
    pos0 = jnp.sum(oh0 * (start + c0 - 1), axis=0, keepdims=True)
    pos1 = jnp.sum(oh1 * (start + tot0 + c1 - 1), axis=0, keepdims=True)
    ri = jax.lax.broadcasted_iota(jnp.int32, (_E, _T), 0)
    pos_ref[...] = jnp.where(ri == 0, pos0, jnp.where(ri == 1, pos1, 0))
    ti = jax.lax.broadcasted_iota(jnp.int32, (_E, _EL), 1)
    te = jnp.sum((ts <= ti).astype(jnp.int32), axis=0, keepdims=True) - 1
    te = jnp.clip(te, 0, _E - 1)
    ntot = jnp.sum(nt, axis=0, keepdims=True)
    ri2 = jax.lax.broadcasted_iota(jnp.int32, (_E, _EL), 0)
    te_ref[...] = jnp.where(ri2 == 1, ntot, te)


def _ridx(i0r, i1r):
    return pl.pallas_call(
        _ridx_body,
        out_shape=[
            jax.ShapeDtypeStruct((_E, _T), jnp.int32),
            jax.ShapeDtypeStruct((_E, _EL), jnp.int32),
        ],
    )(i0r, i1r)


def _sc_dispatch(x_flat, pos_flat):
    mesh = plsc.VectorSubcoreMesh(core_axis_name="c", subcore_axis_name="s")
    tpw = _T // _NW

    @functools.partial(
        pl.kernel,
        out_type=jax.ShapeDtypeStruct((_NP, _H), jnp.float32),
        mesh=mesh,
        scratch_types=[
            pltpu.VMEM((_CH,), jnp.int32),
            pltpu.VMEM((_CH, _H), jnp.float32),
            pltpu.SemaphoreType.DMA,
        ],
    )
    def _k(x_hbm, pos_hbm, xd_hbm, idx_v, rows_v, sem):
        wid = jax.lax.axis_index("s") * 2 + jax.lax.axis_index("c")
        base = wid * tpw
        for k in range(2):
            koff = k * _T

            @pl.loop(0, tpw, step=_CH)
            def _(c):
                b = base + c
                pltpu.sync_copy(pos_hbm.at[pl.ds(koff + b, _CH)], idx_v)
                pltpu.sync_copy(x_hbm.at[pl.ds(b, _CH)], rows_v)
                pltpu.async_copy(rows_v, xd_hbm.at[idx_v], sem).wait()

    return _k(x_flat, pos_flat)


def _gffn_body(te_ref, xd_ref, eg_ref, eu_ref, ed_ref, y_ref):
    h_id = pl.program_id(0)
    j = pl.program_id(1)
    i = pl.program_id(2)
    n_active = te_ref[_EL]

    @pl.when(h_id * _NH + i < n_active)
    def _():
        x = xd_ref[...].astype(jnp.float32)
        g = eg_ref[0]
        u = eu_ref[0]
        d = ed_ref[0]
        a = jax.lax.dot_general(x, g, (((1,), (1,)), ((), ())),
                                preferred_element_type=jnp.float32)
        bb = jax.lax.dot_general(x, u, (((1,), (1,)), ((), ())),
                                 preferred_element_type=jnp.float32)
        h = a * jax.lax.logistic(a) * bb
        yj = jax.lax.dot_general(h, d, (((1,), (1,)), ((), ())),
                                 preferred_element_type=jnp.float32)
        sl = pl.ds(i * _TILE, _TILE)

        @pl.when(j == 0)
        def _():
            y_ref[sl, :] = yj.astype(y_ref.dtype)

        @pl.when(j != 0)
        def _():
            y_ref[sl, :] += yj.astype(y_ref.dtype)


def _gffn(te, xd, eg, eu, ed):
    grid_spec = pltpu.PrefetchScalarGridSpec(
        num_scalar_prefetch=1,
        grid=(_NQ, _NJ, _NH),
        in_specs=[
            pl.BlockSpec((_TILE, _H), lambda h, j, i, te: (h * _NH + i, 0)),
            pl.BlockSpec((1, _IC, _H),
                         lambda h, j, i, te: (te[h * _NH + i], j, 0)),
            pl.BlockSpec((1, _IC, _H),
                         lambda h, j, i, te: (te[h * _NH + i], j, 0)),
            pl.BlockSpec((1, _H, _IC),
                         lambda h, j, i, te: (te[h * _NH + i], 0, j)),
        ],
        out_specs=pl.BlockSpec((_NH * _TILE, _H), lambda h, j, i, te: (h, 0)),
    )
    return pl.pallas_call(
        _gffn_body,
        grid_spec=grid_spec,
        out_shape=jax.ShapeDtypeStruct((_NP, _H), jnp.float32),
        compiler_params=pltpu.CompilerParams(
            dimension_semantics=("arbitrary", "arbitrary", "arbitrary")),
    )(te, xd, eg, eu, ed)


_TH = _T // 2


def _sffn_body(x_ref, sg_ref, su_ref, sd_ref, y_ref):
    j = pl.program_id(0)
    i = pl.program_id(1)
    x = x_ref[pl.ds(i * _TILE, _TILE), :].astype(jnp.float32)
    a = jax.lax.dot_general(x, sg_ref[...], (((1,), (1,)), ((), ())),
                            preferred_element_type=jnp.float32)
    bb = jax.lax.dot_general(x, su_ref[...], (((1,), (1,)), ((), ())),
                             preferred_element_type=jnp.float32)
    h = a * jax.lax.logistic(a) * bb
    yj = jax.lax.dot_general(h, sd_ref[...], (((1,), (1,)), ((), ())),
                             preferred_element_type=jnp.float32)
    sl = pl.ds(i * _TILE, _TILE)

    @pl.when(j == 0)
    def _():
        y_ref[sl, :] = yj

    @pl.when(j != 0)
    def _():
        y_ref[sl, :] += yj


def _sffn_half(x_half, sg, su, sd):
    return pl.pallas_call(
        _sffn_body,
        grid=(_NJ, _TH // _TILE),
        in_specs=[
            pl.BlockSpec((_TH, _H), lambda j, i: (0, 0)),
            pl.BlockSpec((_IC, _H), lambda j, i: (j, 0)),
            pl.BlockSpec((_IC, _H), lambda j, i: (j, 0)),
            pl.BlockSpec((_H, _IC), lambda j, i: (0, j)),
        ],
        out_specs=pl.BlockSpec((_TH, _H), lambda j, i: (0, 0)),
        out_shape=jax.ShapeDtypeStruct((_TH, _H), jnp.float32),
        compiler_params=pltpu.CompilerParams(
            dimension_semantics=("arbitrary", "arbitrary")),
    )(x_half, sg, su, sd)


def _sc_combine(y, pos_flat):
    mesh = plsc.VectorSubcoreMesh(core_axis_name="c", subcore_axis_name="s")
    tpw = _T // _NW

    @functools.partial(
        pl.kernel,
        out_type=[
            jax.ShapeDtypeStruct((_T, _H), jnp.float32),
            jax.ShapeDtypeStruct((_T, _H), jnp.float32),
        ],
        mesh=mesh,
        scratch_types=[
            pltpu.VMEM((_CH,), jnp.int32),
            pltpu.VMEM((_CH, _H), jnp.float32),
            pltpu.SemaphoreType.DMA,
        ],
    )
    def _k(y_hbm, pos_hbm, g0_hbm, g1_hbm, idx_v, rows_v, sem):
        wid = jax.lax.axis_index("s") * 2 + jax.lax.axis_index("c")
        base = wid * tpw
        for k in range(2):
            koff = k * _T
            out = g0_hbm if k == 0 else g1_hbm

            @pl.loop(0, tpw, step=_CH)
            def _(c):
                b = base + c
                pltpu.sync_copy(pos_hbm.at[pl.ds(koff + b, _CH)], idx_v)
                pltpu.async_copy(y_hbm.at[idx_v], rows_v, sem).wait()
                pltpu.sync_copy(rows_v, out.at[pl.ds(b, _CH)])

    return _k(y, pos_flat)


_NFT = _T // _TILE
_NFH = _NFT // 2


def _fin_body(sl_ref, sh_ref, g0_ref, g1_ref, w_ref, o_ref):
    i = pl.program_id(0)
    w0 = w_ref[:, 0:1]
    w1 = w_ref[:, 1:2]
    s = jnp.where(i < _NFH, sl_ref[...], sh_ref[...])
    o_ref[...] = (s
                  + w0 * g0_ref[...].astype(jnp.float32)
                  + w1 * g1_ref[...].astype(jnp.float32))


def _final(sh_lo, sh_hi, g0, g1, wout):
    return pl.pallas_call(
        _fin_body,
        grid=(_NFT,),
        in_specs=[
            pl.BlockSpec((_TILE, _H), lambda i: (jnp.minimum(i, _NFH - 1), 0)),
            pl.BlockSpec((_TILE, _H),
                         lambda i: (jnp.maximum(i - _NFH, 0), 0)),
            pl.BlockSpec((_TILE, _H), lambda i: (i, 0)),
            pl.BlockSpec((_TILE, _H), lambda i: (i, 0)),
            pl.BlockSpec((_TILE, _EL), lambda i: (i, 0)),
        ],
        out_specs=pl.BlockSpec((_TILE, _H), lambda i: (i, 0)),
        out_shape=jax.ShapeDtypeStruct((_T, _H), jnp.float32),
    )(sh_lo, sh_hi, g0, g1, wout)


def kernel(x, ln_gamma, ln_beta, gate_w, eg, eu, ed, sg, su, sd):
    b, s, h = x.shape
    x_flat = x.reshape(-1, h)
    gw_pad = jnp.zeros((_EL, _H), jnp.float32).at[:_E].set(gate_w)
    wout, iout = _router(x_flat, ln_gamma.reshape(1, _H),
                         ln_beta.reshape(1, _H), gw_pad)
    i0r = iout[:, 0].reshape(1, _T)
    i1r = iout[:, 1].reshape(1, _T)
    pos_out, te_out = _ridx(i0r, i1r)
    pos_flat = pos_out[:2].reshape(-1)
    te = te_out[:2].reshape(-1)
    xd = _sc_dispatch(x_flat, pos_flat)
    xb = x_flat.astype(jnp.bfloat16)
    sh_lo = _sffn_half(xb[:_TH], sg, su, sd)
    y = _gffn(te, xd.astype(jnp.bfloat16), eg, eu, ed)
    sh_hi = _sffn_half(xb[_TH:], sg, su, sd)
    g0, g1 = _sc_combine(y, pos_flat)
    out = _final(sh_lo, sh_hi, g0, g1, wout)
    return out.reshape(b, s, h)


# speedup vs baseline: 1.5731x; 1.0154x over previous
import functools

import jax
import jax.numpy as jnp
from jax.experimental import pallas as pl
from jax.experimental.pallas import tpu as pltpu
from jax.experimental.pallas import tpu_sc as plsc

_EPS = 1e-05
_LN_EPS = 1e-05

_T = 4096
_H = 1024
_I = 4096
_E = 8
_EL = 128
_TILE = 512
_NT = 24
_NQ = 4
_NH = _NT // _NQ
_NP = _NT * _TILE
_IC = 1024
_NJ = _I // _IC

_NW = 32
_CH = 32


def _router_body(x_ref, g_ref, b_ref, gw_ref, w_ref, i_ref):
    x = x_ref[...]
    mu = jnp.mean(x, axis=1, keepdims=True)
    xc = x - mu
    var = jnp.mean(xc * xc, axis=1, keepdims=True)
    xn = xc * jax.lax.rsqrt(var + _LN_EPS) * g_ref[...] + b_ref[...]
    logits = jax.lax.dot_general(xn, gw_ref[...], (((1,), (1,)), ((), ())),
                                 preferred_element_type=jnp.float32)
    lane = jax.lax.broadcasted_iota(jnp.int32, logits.shape, 1)
    valid = lane < _E
    logits = jnp.where(valid, logits, -1e30)
    m = jnp.max(logits, axis=1, keepdims=True)
    p = jnp.exp(logits - m)
    p = jnp.where(valid, p, 0.0)
    probs = p / jnp.sum(p, axis=1, keepdims=True)
    p0 = jnp.max(probs, axis=1, keepdims=True)
    i0 = jnp.min(jnp.where(probs >= p0, lane, _EL), axis=1, keepdims=True)
    probs2 = jnp.where(lane == i0, -1.0, probs)
    p1 = jnp.max(probs2, axis=1, keepdims=True)
    i1 = jnp.min(jnp.where(probs2 >= p1, lane, _EL), axis=1, keepdims=True)
    s = p0 + p1 + _EPS
    w0 = p0 / s
    w1 = p1 / s
    w_ref[...] = jnp.where(lane == 0, w0, jnp.where(lane == 1, w1, 0.0))
    i_ref[...] = jnp.where(lane == 0, i0, jnp.where(lane == 1, i1, 0))


def _router(x_flat, gamma, beta, gw_pad):
    tt = 512
    return pl.pallas_call(
        _router_body,
        grid=(_T // tt,),
        in_specs=[
            pl.BlockSpec((tt, _H), lambda i: (i, 0)),
            pl.BlockSpec((1, _H), lambda i: (0, 0)),
            pl.BlockSpec((1, _H), lambda i: (0, 0)),
            pl.BlockSpec((_EL, _H), lambda i: (0, 0)),
        ],
        out_specs=[
            pl.BlockSpec((tt, _EL), lambda i: (i, 0)),
            pl.BlockSpec((tt, _EL), lambda i: (i, 0)),
        ],
        out_shape=[
            jax.ShapeDtypeStruct((_T, _EL), jnp.float32),
            jax.ShapeDtypeStruct((_T, _EL), jnp.int32),
        ],
    )(x_flat, gamma, beta, gw_pad)


def _ridx_body(i0_ref, i1_ref, pos_ref, te_ref):
    erow = jax.lax.broadcasted_iota(jnp.int32, (_E, _T), 0)
    oh0 = (i0_ref[...] == erow).astype(jnp.int32)
    oh1 = (i1_ref[...] == erow).astype(jnp.int32)
    li = jax.lax.broadcasted_iota(jnp.int32, (_E, _T), 1)

    def lane_cumsum(a):
        s = 1
        while s < _T:
            sh = pltpu.roll(a, s, axis=1)
            a = a + jnp.where(li >= s, sh, 0)
            s *= 2
        return a

    c0 = lane_cumsum(oh0)
    c1 = lane_cumsum(oh1)
    tot0 = jnp.sum(oh0, axis=1, keepdims=True)
    counts = tot0 + jnp.sum(oh1, axis=1, keepdims=True)
    nt = (counts + (_TILE - 1)) // _TILE
    inc = nt
    s = 1
    while s < _E:
        inc = inc + jnp.concatenate(
            [jnp.zeros((s, 1), jnp.int32), inc[:-s]], axis=0)
        s *= 2
    ts = inc - nt
    start = ts * _TILE
    pos0 = jnp.sum(oh0 * (start + c0 - 1), axis=0, keepdims=True)
    pos1 = jnp.sum(oh1 * (start + tot0 + c1 - 1), axis=0, keepdims=True)
    ri = jax.lax.broadcasted_iota(jnp.int32, (_E, _T), 0)
    pos_ref[...] = jnp.where(ri == 0, pos0, jnp.where(ri == 1, pos1, 0))
    ti = jax.lax.broadcasted_iota(jnp.int32, (_E, _EL), 1)
    te = jnp.sum((ts <= ti).astype(jnp.int32), axis=0, keepdims=True) - 1
    te = jnp.clip(te, 0, _E - 1)
    ntot = jnp.sum(nt, axis=0, keepdims=True)
    ri2 = jax.lax.broadcasted_iota(jnp.int32, (_E, _EL), 0)
    te_ref[...] = jnp.where(ri2 == 1, ntot, te)


def _ridx(i0r, i1r):
    return pl.pallas_call(
        _ridx_body,
        out_shape=[
            jax.ShapeDtypeStruct((_E, _T), jnp.int32),
            jax.ShapeDtypeStruct((_E, _EL), jnp.int32),
        ],
    )(i0r, i1r)


def _sc_dispatch(x_flat, pos_flat):
    mesh = plsc.VectorSubcoreMesh(core_axis_name="c", subcore_axis_name="s")
    tpw = _T // _NW

    @functools.partial(
        pl.kernel,
        out_type=jax.ShapeDtypeStruct((_NP, _H), jnp.float32),
        mesh=mesh,
        scratch_types=[
            pltpu.VMEM((_CH,), jnp.int32),
            pltpu.VMEM((_CH, _H), jnp.float32),
            pltpu.SemaphoreType.DMA,
        ],
    )
    def _k(x_hbm, pos_hbm, xd_hbm, idx_v, rows_v, sem):
        wid = jax.lax.axis_index("s") * 2 + jax.lax.axis_index("c")
        base = wid * tpw
        for k in range(2):
            koff = k * _T

            @pl.loop(0, tpw, step=_CH)
            def _(c):
                b = base + c
                pltpu.sync_copy(pos_hbm.at[pl.ds(koff + b, _CH)], idx_v)
                pltpu.sync_copy(x_hbm.at[pl.ds(b, _CH)], rows_v)
                pltpu.async_copy(rows_v, xd_hbm.at[idx_v], sem).wait()

    return _k(x_flat, pos_flat)


def _gffn_body(te_ref, xd_ref, eg_ref, eu_ref, ed_ref, y_ref):
    h_id = pl.program_id(0)
    j = pl.program_id(1)
    i = pl.program_id(2)
    n_active = te_ref[_EL]

    @pl.when(h_id * _NH + i < n_active)
    def _():
        x = xd_ref[...]
        g = eg_ref[0]
        u = eu_ref[0]
        d = ed_ref[0]
        a = jax.lax.dot_general(x, g, (((1,), (1,)), ((), ())),
                                preferred_element_type=jnp.float32)
        bb = jax.lax.dot_general(x, u, (((1,), (1,)), ((), ())),
                                 preferred_element_type=jnp.float32)
        h = a * jax.lax.logistic(a) * bb
        yj = jax.lax.dot_general(h, d, (((1,), (1,)), ((), ())),
                                 preferred_element_type=jnp.float32)
        sl = pl.ds(i * _TILE, _TILE)

        @pl.when(j == 0)
        def _():
            y_ref[sl, :] = yj.astype(y_ref.dtype)

        @pl.when(j != 0)
        def _():
            y_ref[sl, :] += yj.astype(y_ref.dtype)


def _gffn(te, xd, eg, eu, ed):
    grid_spec = pltpu.PrefetchScalarGridSpec(
        num_scalar_prefetch=1,
        grid=(_NQ, _NJ, _NH),
        in_specs=[
            pl.BlockSpec((_TILE, _H), lambda h, j, i, te: (h * _NH + i, 0)),
            pl.BlockSpec((1, _IC, _H),
                         lambda h, j, i, te: (te[h * _NH + i], j, 0)),
            pl.BlockSpec((1, _IC, _H),
                         lambda h, j, i, te: (te[h * _NH + i], j, 0)),
            pl.BlockSpec((1, _H, _IC),
                         lambda h, j, i, te: (te[h * _NH + i], 0, j)),
        ],
        out_specs=pl.BlockSpec((_NH * _TILE, _H), lambda h, j, i, te: (h, 0)),
    )
    return pl.pallas_call(
        _gffn_body,
        grid_spec=grid_spec,
        out_shape=jax.ShapeDtypeStruct((_NP, _H), jnp.float32),
        compiler_params=pltpu.CompilerParams(
            dimension_semantics=("arbitrary", "arbitrary", "arbitrary")),
    )(te, xd, eg, eu, ed)


_TH = _T // 2


def _sffn_body(x_ref, sg_ref, su_ref, sd_ref, y_ref):
    j = pl.program_id(0)
    i = pl.program_id(1)
    x = x_ref[pl.ds(i * _TILE, _TILE), :].astype(jnp.float32)
    a = jax.lax.dot_general(x, sg_ref[...], (((1,), (1,)), ((), ())),
                            preferred_element_type=jnp.float32)
    bb = jax.lax.dot_general(x, su_ref[...], (((1,), (1,)), ((), ())),
                             preferred_element_type=jnp.float32)
    h = a * jax.lax.logistic(a) * bb
    yj = jax.lax.dot_general(h, sd_ref[...], (((1,), (1,)), ((), ())),
                             preferred_element_type=jnp.float32)
    sl = pl.ds(i * _TILE, _TILE)

    @pl.when(j == 0)
    def _():
        y_ref[sl, :] = yj

    @pl.when(j != 0)
    def _():
        y_ref[sl, :] += yj


def _sffn_half(x_half, sg, su, sd):
    return pl.pallas_call(
        _sffn_body,
        grid=(_NJ, _TH // _TILE),
        in_specs=[
            pl.BlockSpec((_TH, _H), lambda j, i: (0, 0)),
            pl.BlockSpec((_IC, _H), lambda j, i: (j, 0)),
            pl.BlockSpec((_IC, _H), lambda j, i: (j, 0)),
            pl.BlockSpec((_H, _IC), lambda j, i: (0, j)),
        ],
        out_specs=pl.BlockSpec((_TH, _H), lambda j, i: (0, 0)),
        out_shape=jax.ShapeDtypeStruct((_TH, _H), jnp.float32),
        compiler_params=pltpu.CompilerParams(
            dimension_semantics=("arbitrary", "arbitrary")),
    )(x_half, sg, su, sd)


def _sc_combine(y, pos_flat):
    mesh = plsc.VectorSubcoreMesh(core_axis_name="c", subcore_axis_name="s")
    tpw = _T // _NW

    @functools.partial(
        pl.kernel,
        out_type=[
            jax.ShapeDtypeStruct((_T, _H), jnp.float32),
            jax.ShapeDtypeStruct((_T, _H), jnp.float32),
        ],
        mesh=mesh,
        scratch_types=[
            pltpu.VMEM((_CH,), jnp.int32),
            pltpu.VMEM((_CH, _H), jnp.float32),
            pltpu.SemaphoreType.DMA,
        ],
    )
    def _k(y_hbm, pos_hbm, g0_hbm, g1_hbm, idx_v, rows_v, sem):
        wid = jax.lax.axis_index("s") * 2 + jax.lax.axis_index("c")
        base = wid * tpw
        for k in range(2):
            koff = k * _T
            out = g0_hbm if k == 0 else g1_hbm

            @pl.loop(0, tpw, step=_CH)
            def _(c):
                b = base + c
                pltpu.sync_copy(pos_hbm.at[pl.ds(koff + b, _CH)], idx_v)
                pltpu.async_copy(y_hbm.at[idx_v], rows_v, sem).wait()
                pltpu.sync_copy(rows_v, out.at[pl.ds(b, _CH)])

    return _k(y, pos_flat)


_NFT = _T // _TILE
_NFH = _NFT // 2


def _fin_body(sl_ref, sh_ref, g0_ref, g1_ref, w_ref, o_ref):
    i = pl.program_id(0)
    w0 = w_ref[:, 0:1]
    w1 = w_ref[:, 1:2]
    s = jnp.where(i < _NFH, sl_ref[...], sh_ref[...])
    o_ref[...] = (s
                  + w0 * g0_ref[...].astype(jnp.float32)
                  + w1 * g1_ref[...].astype(jnp.float32))


def _final(sh_lo, sh_hi, g0, g1, wout):
    return pl.pallas_call(
        _fin_body,
        grid=(_NFT,),
        in_specs=[
            pl.BlockSpec((_TILE, _H), lambda i: (jnp.minimum(i, _NFH - 1), 0)),
            pl.BlockSpec((_TILE, _H),
                         lambda i: (jnp.maximum(i - _NFH, 0), 0)),
            pl.BlockSpec((_TILE, _H), lambda i: (i, 0)),
            pl.BlockSpec((_TILE, _H), lambda i: (i, 0)),
            pl.BlockSpec((_TILE, _EL), lambda i: (i, 0)),
        ],
        out_specs=pl.BlockSpec((_TILE, _H), lambda i: (i, 0)),
        out_shape=jax.ShapeDtypeStruct((_T, _H), jnp.float32),
    )(sh_lo, sh_hi, g0, g1, wout)


def kernel(x, ln_gamma, ln_beta, gate_w, eg, eu, ed, sg, su, sd):
    b, s, h = x.shape
    x_flat = x.reshape(-1, h)
    gw_pad = jnp.zeros((_EL, _H), jnp.float32).at[:_E].set(gate_w)
    wout, iout = _router(x_flat, ln_gamma.reshape(1, _H),
                         ln_beta.reshape(1, _H), gw_pad)
    i0r = iout[:, 0].reshape(1, _T)
    i1r = iout[:, 1].reshape(1, _T)
    pos_out, te_out = _ridx(i0r, i1r)
    pos_flat = pos_out[:2].reshape(-1)
    te = te_out[:2].reshape(-1)
    xb = x_flat.astype(jnp.bfloat16)
    xd = _sc_dispatch(x_flat, pos_flat)
    sh_lo = _sffn_half(xb[:_TH], sg, su, sd)
    y = _gffn(te, xd, eg, eu, ed)
    sh_hi = _sffn_half(xb[_TH:], sg, su, sd)
    g0, g1 = _sc_combine(y, pos_flat)
    out = _final(sh_lo, sh_hi, g0, g1, wout)
    return out.reshape(b, s, h)


# speedup vs baseline: 1.5941x; 1.0134x over previous
import functools

import jax
import jax.numpy as jnp
from jax.experimental import pallas as pl
from jax.experimental.pallas import tpu as pltpu
from jax.experimental.pallas import tpu_sc as plsc

_EPS = 1e-05
_LN_EPS = 1e-05

_T = 4096
_H = 1024
_I = 4096
_E = 8
_EL = 128
_TILE = 512
_NT = 24
_NQ = 4
_NH = _NT // _NQ
_NP = _NT * _TILE
_IC = 1024
_NJ = _I // _IC

_NW = 32
_CH = 64


def _router_body(x_ref, g_ref, b_ref, gw_ref, w_ref, i_ref):
    x = x_ref[...]
    mu = jnp.mean(x, axis=1, keepdims=True)
    xc = x - mu
    var = jnp.mean(xc * xc, axis=1, keepdims=True)
    xn = xc * jax.lax.rsqrt(var + _LN_EPS) * g_ref[...] + b_ref[...]
    logits = jax.lax.dot_general(xn, gw_ref[...], (((1,), (1,)), ((), ())),
                                 preferred_element_type=jnp.float32)
    lane = jax.lax.broadcasted_iota(jnp.int32, logits.shape, 1)
    valid = lane < _E
    logits = jnp.where(valid, logits, -1e30)
    m = jnp.max(logits, axis=1, keepdims=True)
    p = jnp.exp(logits - m)
    p = jnp.where(valid, p, 0.0)
    probs = p / jnp.sum(p, axis=1, keepdims=True)
    p0 = jnp.max(probs, axis=1, keepdims=True)
    i0 = jnp.min(jnp.where(probs >= p0, lane, _EL), axis=1, keepdims=True)
    probs2 = jnp.where(lane == i0, -1.0, probs)
    p1 = jnp.max(probs2, axis=1, keepdims=True)
    i1 = jnp.min(jnp.where(probs2 >= p1, lane, _EL), axis=1, keepdims=True)
    s = p0 + p1 + _EPS
    w0 = p0 / s
    w1 = p1 / s
    w_ref[...] = jnp.where(lane == 0, w0, jnp.where(lane == 1, w1, 0.0))
    i_ref[...] = jnp.where(lane == 0, i0, jnp.where(lane == 1, i1, 0))


def _router(x_flat, gamma, beta, gw_pad):
    tt = 512
    return pl.pallas_call(
        _router_body,
        grid=(_T // tt,),
        in_specs=[
            pl.BlockSpec((tt, _H), lambda i: (i, 0)),
            pl.BlockSpec((1, _H), lambda i: (0, 0)),
            pl.BlockSpec((1, _H), lambda i: (0, 0)),
            pl.BlockSpec((_EL, _H), lambda i: (0, 0)),
        ],
        out_specs=[
            pl.BlockSpec((tt, _EL), lambda i: (i, 0)),
            pl.BlockSpec((tt, _EL), lambda i: (i, 0)),
        ],
        out_shape=[
            jax.ShapeDtypeStruct((_T, _EL), jnp.float32),
            jax.ShapeDtypeStruct((_T, _EL), jnp.int32),
        ],
    )(x_flat, gamma, beta, gw_pad)


def _ridx_body(i0_ref, i1_ref, pos_ref, te_ref):
    erow = jax.lax.broadcasted_iota(jnp.int32, (_E, _T), 0)
    oh0 = (i0_ref[...] == erow).astype(jnp.int32)
    oh1 = (i1_ref[...] == erow).astype(jnp.int32)
    li = jax.lax.broadcasted_iota(jnp.int32, (_E, _T), 1)

    def lane_cumsum(a):
        s = 1
        while s < _T:
            sh = pltpu.roll(a, s, axis=1)
            a = a + jnp.where(li >= s, sh, 0)
            s *= 2
        return a

    c0 = lane_cumsum(oh0)
    c1 = lane_cumsum(oh1)
    tot0 = jnp.sum(oh0, axis=1, keepdims=True)
    counts = tot0 + jnp.sum(oh1, axis=1, keepdims=True)
    nt = (counts + (_TILE - 1)) // _TILE
    inc = nt
    s = 1
    while s < _E:
        inc = inc + jnp.concatenate(
            [jnp.zeros((s, 1), jnp.int32), inc[:-s]], axis=0)
        s *= 2
    ts = inc - nt
    start = ts * _TILE
    pos0 = jnp.sum(oh0 * (start + c0 - 1), axis=0, keepdims=True)
    pos1 = jnp.sum(oh1 * (start + tot0 + c1 - 1), axis=0, keepdims=True)
    ri = jax.lax.broadcasted_iota(jnp.int32, (_E, _T), 0)
    pos_ref[...] = jnp.where(ri == 0, pos0, jnp.where(ri == 1, pos1, 0))
    ti = jax.lax.broadcasted_iota(jnp.int32, (_E, _EL), 1)
    te = jnp.sum((ts <= ti).astype(jnp.int32), axis=0, keepdims=True) - 1
    te = jnp.clip(te, 0, _E - 1)
    ntot = jnp.sum(nt, axis=0, keepdims=True)
    ri2 = jax.lax.broadcasted_iota(jnp.int32, (_E, _EL), 0)
    te_ref[...] = jnp.where(ri2 == 1, ntot, te)


def _ridx(i0r, i1r):
    return pl.pallas_call(
        _ridx_body,
        out_shape=[
            jax.ShapeDtypeStruct((_E, _T), jnp.int32),
            jax.ShapeDtypeStruct((_E, _EL), jnp.int32),
        ],
    )(i0r, i1r)


def _sc_dispatch(x_flat, pos_flat):
    mesh = plsc.VectorSubcoreMesh(core_axis_name="c", subcore_axis_name="s")
    tpw = _T // _NW

    @functools.partial(
        pl.kernel,
        out_type=jax.ShapeDtypeStruct((_NP, _H), jnp.float32),
        mesh=mesh,
        scratch_types=[
            pltpu.VMEM((2 * (128 // _CH), _CH), jnp.int32),
            pltpu.VMEM((_CH, _H), jnp.float32),
            pltpu.SemaphoreType.DMA,
            pltpu.SemaphoreType.DMA,
        ],
    )
    def _k(x_hbm, pos_hbm, xd_hbm, idx_v, rows_v, lsem, ssem):
        wid = jax.lax.axis_index("s") * 2 + jax.lax.axis_index("c")
        base = wid * tpw
        nch = tpw // _CH
        for k in range(2):
            for c in range(nch):
                pltpu.sync_copy(
                    pos_hbm.at[pl.ds(k * _T + base + c * _CH, _CH)],
                    idx_v.at[k * nch + c])
        for c in range(nch):
            pltpu.async_copy(x_hbm.at[pl.ds(base + c * _CH, _CH)],
                             rows_v, lsem).wait()
            s0 = pltpu.async_copy(rows_v, xd_hbm.at[idx_v.at[c]], ssem)
            s1 = pltpu.async_copy(rows_v, xd_hbm.at[idx_v.at[nch + c]], ssem)
            s0.wait()
            s1.wait()

    return _k(x_flat, pos_flat)


def _gffn_body(te_ref, xd_ref, eg_ref, eu_ref, ed_ref, y_ref):
    h_id = pl.program_id(0)
    j = pl.program_id(1)
    i = pl.program_id(2)
    n_active = te_ref[_EL]

    @pl.when(h_id * _NH + i < n_active)
    def _():
        x = xd_ref[...]
        g = eg_ref[0]
        u = eu_ref[0]
        d = ed_ref[0]
        a = jax.lax.dot_general(x, g, (((1,), (1,)), ((), ())),
                                preferred_element_type=jnp.float32)
        bb = jax.lax.dot_general(x, u, (((1,), (1,)), ((), ())),
                                 preferred_element_type=jnp.float32)
        h = a * jax.lax.logistic(a) * bb
        yj = jax.lax.dot_general(h, d, (((1,), (1,)), ((), ())),
                                 preferred_element_type=jnp.float32)
        sl = pl.ds(i * _TILE, _TILE)

        @pl.when(j == 0)
        def _():
            y_ref[sl, :] = yj.astype(y_ref.dtype)

        @pl.when(j != 0)
        def _():
            y_ref[sl, :] += yj.astype(y_ref.dtype)


def _gffn(te, xd, eg, eu, ed):
    grid_spec = pltpu.PrefetchScalarGridSpec(
        num_scalar_prefetch=1,
        grid=(_NQ, _NJ, _NH),
        in_specs=[
            pl.BlockSpec((_TILE, _H), lambda h, j, i, te: (h * _NH + i, 0)),
            pl.BlockSpec((1, _IC, _H),
                         lambda h, j, i, te: (te[h * _NH + i], j, 0)),
            pl.BlockSpec((1, _IC, _H),
                         lambda h, j, i, te: (te[h * _NH + i], j, 0)),
            pl.BlockSpec((1, _H, _IC),
                         lambda h, j, i, te: (te[h * _NH + i], 0, j)),
        ],
        out_specs=pl.BlockSpec((_NH * _TILE, _H), lambda h, j, i, te: (h, 0)),
    )
    return pl.pallas_call(
        _gffn_body,
        grid_spec=grid_spec,
        out_shape=jax.ShapeDtypeStruct((_NP, _H), jnp.float32),
        compiler_params=pltpu.CompilerParams(
            dimension_semantics=("arbitrary", "arbitrary", "arbitrary")),
    )(te, xd, eg, eu, ed)


_TH = _T // 2


def _sffn_body(x_ref, sg_ref, su_ref, sd_ref, y_ref):
    j = pl.program_id(0)
    i = pl.program_id(1)
    x = x_ref[pl.ds(i * _TILE, _TILE), :].astype(jnp.float32)
    a = jax.lax.dot_general(x, sg_ref[...], (((1,), (1,)), ((), ())),
                            preferred_element_type=jnp.float32)
    bb = jax.lax.dot_general(x, su_ref[...], (((1,), (1,)), ((), ())),
                             preferred_element_type=jnp.float32)
    h = a * jax.lax.logistic(a) * bb
    yj = jax.lax.dot_general(h, sd_ref[...], (((1,), (1,)), ((), ())),
                             preferred_element_type=jnp.float32)
    sl = pl.ds(i * _TILE, _TILE)

    @pl.when(j == 0)
    def _():
        y_ref[sl, :] = yj

    @pl.when(j != 0)
    def _():
        y_ref[sl, :] += yj


def _sffn_half(x_half, sg, su, sd):
    return pl.pallas_call(
        _sffn_body,
        grid=(_NJ, _TH // _TILE),
        in_specs=[
            pl.BlockSpec((_TH, _H), lambda j, i: (0, 0)),
            pl.BlockSpec((_IC, _H), lambda j, i: (j, 0)),
            pl.BlockSpec((_IC, _H), lambda j, i: (j, 0)),
            pl.BlockSpec((_H, _IC), lambda j, i: (0, j)),
        ],
        out_specs=pl.BlockSpec((_TH, _H), lambda j, i: (0, 0)),
        out_shape=jax.ShapeDtypeStruct((_TH, _H), jnp.float32),
        compiler_params=pltpu.CompilerParams(
            dimension_semantics=("arbitrary", "arbitrary")),
    )(x_half, sg, su, sd)


def _sc_combine(y, pos_flat):
    mesh = plsc.VectorSubcoreMesh(core_axis_name="c", subcore_axis_name="s")
    tpw = _T // _NW

    @functools.partial(
        pl.kernel,
        out_type=[
            jax.ShapeDtypeStruct((_T, _H), jnp.float32),
            jax.ShapeDtypeStruct((_T, _H), jnp.float32),
        ],
        mesh=mesh,
        scratch_types=[
            pltpu.VMEM((_CH,), jnp.int32),
            pltpu.VMEM((_CH, _H), jnp.float32),
            pltpu.SemaphoreType.DMA,
        ],
    )
    def _k(y_hbm, pos_hbm, g0_hbm, g1_hbm, idx_v, rows_v, sem):
        wid = jax.lax.axis_index("s") * 2 + jax.lax.axis_index("c")
        base = wid * tpw
        for k in range(2):
            koff = k * _T
            out = g0_hbm if k == 0 else g1_hbm

            @pl.loop(0, tpw, step=_CH)
            def _(c):
                b = base + c
                pltpu.sync_copy(pos_hbm.at[pl.ds(koff + b, _CH)], idx_v)
                pltpu.async_copy(y_hbm.at[idx_v], rows_v, sem).wait()
                pltpu.sync_copy(rows_v, out.at[pl.ds(b, _CH)])

    return _k(y, pos_flat)


_NFT = _T // _TILE
_NFH = _NFT // 2


def _fin_body(sl_ref, sh_ref, g0_ref, g1_ref, w_ref, o_ref):
    i = pl.program_id(0)
    w0 = w_ref[:, 0:1]
    w1 = w_ref[:, 1:2]
    s = jnp.where(i < _NFH, sl_ref[...], sh_ref[...])
    o_ref[...] = (s
                  + w0 * g0_ref[...].astype(jnp.float32)
                  + w1 * g1_ref[...].astype(jnp.float32))


def _final(sh_lo, sh_hi, g0, g1, wout):
    return pl.pallas_call(
        _fin_body,
        grid=(_NFT,),
        in_specs=[
            pl.BlockSpec((_TILE, _H), lambda i: (jnp.minimum(i, _NFH - 1), 0)),
            pl.BlockSpec((_TILE, _H),
                         lambda i: (jnp.maximum(i - _NFH, 0), 0)),
            pl.BlockSpec((_TILE, _H), lambda i: (i, 0)),
            pl.BlockSpec((_TILE, _H), lambda i: (i, 0)),
            pl.BlockSpec((_TILE, _EL), lambda i: (i, 0)),
        ],
        out_specs=pl.BlockSpec((_TILE, _H), lambda i: (i, 0)),
        out_shape=jax.ShapeDtypeStruct((_T, _H), jnp.float32),
    )(sh_lo, sh_hi, g0, g1, wout)


def kernel(x, ln_gamma, ln_beta, gate_w, eg, eu, ed, sg, su, sd):
    b, s, h = x.shape
    x_flat = x.reshape(-1, h)
    gw_pad = jnp.zeros((_EL, _H), jnp.float32).at[:_E].set(gate_w)
    wout, iout = _router(x_flat, ln_gamma.reshape(1, _H),
                         ln_beta.reshape(1, _H), gw_pad)
    i0r = iout[:, 0].reshape(1, _T)
    i1r = iout[:, 1].reshape(1, _T)
    pos_out, te_out = _ridx(i0r, i1r)
    pos_flat = pos_out[:2].reshape(-1)
    te = te_out[:2].reshape(-1)
    xb = x_flat.astype(jnp.bfloat16)
    xd = _sc_dispatch(x_flat, pos_flat)
    sh_lo = _sffn_half(xb[:_TH], sg, su, sd)
    y = _gffn(te, xd, eg, eu, ed)
    sh_hi = _sffn_half(xb[_TH:], sg, su, sd)
    g0, g1 = _sc_combine(y, pos_flat)
    out = _final(sh_lo, sh_hi, g0, g1, wout)
    return out.reshape(b, s, h)
